# baseline (speedup 1.0000x reference)
import functools

import jax
import jax.numpy as jnp
from jax import lax
from jax.experimental import pallas as pl
from jax.experimental.pallas import tpu as pltpu
from jax.experimental.pallas import tpu_sc as plsc

N = 10000
NP = 10240
E = 320000
EP2 = 327680
NC = 2
NS = 16
NW = NC * NS
EPT = EP2 // NW
CH = 64
NCHT = EPT // CH
NCHR = NCHT // 4
NBUF = 4
PT = NP // NS
BN = 2048
GN = NP // BN
F = 128

def _deg_sc_body(col_hbm, out_hbm, colv, hist, shared, accin, outv):
    c = lax.axis_index("c")
    s = lax.axis_index("s")
    w = c * NS + s
    pltpu.sync_copy(col_hbm.at[pl.ds(w * NCHT, NCHT)], colv)

    def zero_body(i, _):
        hist[pl.ds(i * 16, 16)] = jnp.zeros((16,), jnp.float32)
        return 0
    lax.fori_loop(0, NP // 16, zero_body, 0)

    ones = jnp.ones((16,), jnp.float32)

    def hist_body(i, _):
        r = i // (CH // 16)
        k = (i % (CH // 16)) * 16
        idx = colv[r, pl.ds(k, 16)]
        plsc.addupdate_scatter(hist, [idx], ones)
        return 0
    lax.fori_loop(0, EPT // 16, hist_body, 0)

    pltpu.sync_copy(hist, shared.at[s])
    plsc.subcore_barrier()
    pltpu.sync_copy(shared.at[:, pl.ds(s * PT, PT)], accin)

    def red_body(k, _):
        acc = accin[0, pl.ds(k * 16, 16)]
        for t in range(1, NS):
            acc = acc + accin[t, pl.ds(k * 16, 16)]
        outv[pl.ds(k * 16, 16)] = acc
        return 0
    lax.fori_loop(0, PT // 16, red_body, 0)
    pltpu.sync_copy(outv, out_hbm.at[c, pl.ds(s * PT, PT)])


def _agg_pass(table_hbm, row_hbm, col_hbm, zeros_hbm, dump,
              rowi, coli, bufs, acc, gsems, ssems, s, w):
    pltpu.sync_copy(zeros_hbm.at[pl.ds(s * PT, PT)], acc.at[pl.ds(s * PT, PT)])
    plsc.subcore_barrier()

    def gstart(jj, b):
        pltpu.async_copy(table_hbm.at[rowi.at[jj]], bufs[b], gsems[b])

    def gwait(jj, b):
        pltpu.make_async_copy(table_hbm.at[rowi.at[jj]], bufs[b],
                              gsems[b]).wait()

    def sstart(jj, b):
        pltpu.async_copy(bufs[b], acc.at[coli.at[jj]], ssems[b], add=True)

    def swait(jj, b):
        pltpu.make_async_copy(bufs[b], acc.at[coli.at[jj]], ssems[b]).wait()

    for h in range(NCHT // NCHR):
        base = w * NCHT + h * NCHR
        pltpu.sync_copy(row_hbm.at[pl.ds(base, NCHR)], rowi)
        pltpu.sync_copy(col_hbm.at[pl.ds(base, NCHR)], coli)
        for b in range(NBUF):
            gstart(b, b)

        def body(i, _):
            for b in range(NBUF):
                jj = NBUF * i + b
                gwait(jj, b)
                sstart(jj, b)
            for b in range(NBUF):
                jj = NBUF * i + b

                @pl.when(jj + NBUF < NCHR)
                def _():
                    swait(jj, b)
                    gstart(jj + NBUF, b)
            return 0
        lax.fori_loop(0, NCHR // NBUF, body, 0)
        for b in range(NBUF):
            swait(NCHR - NBUF + b, b)

    plsc.subcore_barrier()
    dump()


def _unpack_scratch(rest):
    bufs = rest[:NBUF]
    acc = rest[NBUF]
    gsems = rest[NBUF + 1:2 * NBUF + 1]
    ssems = rest[2 * NBUF + 1:]
    return bufs, acc, gsems, ssems


def _agg_sc_body(table_hbm, row_hbm, col_hbm, zeros_hbm, out_hbm,
                 rowi, coli, *rest):
    c = lax.axis_index("c")
    s = lax.axis_index("s")
    w = c * NS + s
    bufs, acc, gsems, ssems = _unpack_scratch(rest)

    def dump():
        pltpu.sync_copy(acc.at[pl.ds(s * PT, PT)],
                        out_hbm.at[c, pl.ds(s * PT, PT)])
    _agg_pass(table_hbm, row_hbm, col_hbm, zeros_hbm, dump,
              rowi, coli, bufs, acc, gsems, ssems, s, w)


def _agg2_sc_body(table_hbm, row0_hbm, row1_hbm, col_hbm, zeros_hbm, out_hbm,
                  rowi, coli, *rest):
    c = lax.axis_index("c")
    s = lax.axis_index("s")
    w = c * NS + s
    bufs, acc, gsems, ssems = _unpack_scratch(rest)

    for f, row_hbm in enumerate((row0_hbm, row1_hbm)):
        def dump(f=f):
            pltpu.sync_copy(acc.at[pl.ds(s * PT, PT)],
                            out_hbm.at[f, c, pl.ds(s * PT, PT)])
        _agg_pass(table_hbm, row_hbm, col_hbm, zeros_hbm, dump,
                  rowi, coli, bufs, acc, gsems, ssems, s, w)


@functools.lru_cache(maxsize=None)
def _sc_kernels():
    mesh = plsc.VectorSubcoreMesh(
        core_axis_name="c", subcore_axis_name="s",
        num_cores=NC, num_subcores=NS)
    params = pltpu.CompilerParams(needs_layout_passes=False)
    deg = pl.kernel(
        _deg_sc_body,
        out_type=jax.ShapeDtypeStruct((NC, NP), jnp.float32),
        mesh=mesh,
        compiler_params=params,
        scratch_types=[
            pltpu.VMEM((NCHT, CH), jnp.int32),
            pltpu.VMEM((NP,), jnp.float32),
            pltpu.VMEM_SHARED((NS, NP), jnp.float32),
            pltpu.VMEM((NS, PT), jnp.float32),
            pltpu.VMEM((PT,), jnp.float32),
        ],
    )
    agg_scratch = (
        [
            pltpu.VMEM((NCHR, CH), jnp.int32),
            pltpu.VMEM((NCHR, CH), jnp.int32),
        ]
        + [pltpu.VMEM((CH, F), jnp.float32)] * NBUF
        + [pltpu.VMEM_SHARED((NP, F), jnp.float32)]
        + [pltpu.SemaphoreType.DMA] * (2 * NBUF)
    )
    agg = pl.kernel(
        _agg_sc_body,
        out_type=jax.ShapeDtypeStruct((NC, NP, F), jnp.float32),
        mesh=mesh,
        compiler_params=params,
        scratch_types=agg_scratch,
    )
    agg2 = pl.kernel(
        _agg2_sc_body,
        out_type=jax.ShapeDtypeStruct((2, NC, NP, F), jnp.float32),
        mesh=mesh,
        compiler_params=params,
        scratch_types=agg_scratch,
    )
    return deg, agg, agg2


_SLOPE = (1.0 / 8.0 + 1.0 / 3.0) / 2.0


def _tc_front_body(deg_ref, x_ref, w1_ref, hp_ref, dinv_ref):
    deg = deg_ref[0] + deg_ref[1] + 1.0
    dinv = lax.rsqrt(deg)
    h = jnp.dot(x_ref[...], w1_ref[...], preferred_element_type=jnp.float32)
    hp = h * dinv
    hp_ref[0] = hp[:, :F]
    hp_ref[1] = hp[:, F:]
    dinv_ref[...] = dinv


def _tc_mid_body(a_ref, hp_ref, dinv_ref, b1_ref, w2_ref, h2_ref):
    dinv = dinv_ref[...]
    u0 = dinv * (a_ref[0, 0] + a_ref[0, 1] + hp_ref[0]) + b1_ref[0:1, :]
    u1 = dinv * (a_ref[1, 0] + a_ref[1, 1] + hp_ref[1]) + b1_ref[1:2, :]
    act0 = jnp.where(u0 >= 0, u0, _SLOPE * u0)
    act1 = jnp.where(u1 >= 0, u1, _SLOPE * u1)
    h2 = (jnp.dot(act0, w2_ref[:F, :], preferred_element_type=jnp.float32)
          + jnp.dot(act1, w2_ref[F:, :], preferred_element_type=jnp.float32))
    h2_ref[...] = h2 * dinv


def _tc_back_body(a2_ref, h2_ref, dinv_ref, b2_ref, out_ref):
    out_ref[...] = (dinv_ref[...] * (a2_ref[0] + a2_ref[1] + h2_ref[...])
                    + b2_ref[0:1, :])


def kernel(x, edge_index, W1, b1, W2, b2):
    f32 = jnp.float32
    _deg_sc, _agg_sc, _agg2_sc = _sc_kernels()
    xp = jnp.pad(x, ((0, NP - N), (0, 0)))
    epad = N + (jnp.arange(EP2 - E, dtype=jnp.int32) % (NP - N))
    rowcat = jnp.concatenate([edge_index[0], epad])
    row2 = rowcat.reshape(EP2 // CH, CH)
    row2b = (rowcat + NP).reshape(EP2 // CH, CH)
    col2 = jnp.concatenate([edge_index[1], epad]).reshape(EP2 // CH, CH)
    zeros = jnp.zeros((NP, F), f32)

    degp = _deg_sc(col2)
    degp3 = degp.reshape(NC, NP, 1)

    h1p, dinv = pl.pallas_call(
        _tc_front_body,
        grid=(GN,),
        in_specs=[
            pl.BlockSpec((NC, BN, 1), lambda n: (0, n, 0)),
            pl.BlockSpec((BN, 128), lambda n: (n, 0)),
            pl.BlockSpec((128, 256), lambda n: (0, 0)),
        ],
        out_specs=[
            pl.BlockSpec((2, BN, F), lambda n: (0, n, 0)),
            pl.BlockSpec((BN, 1), lambda n: (n, 0)),
        ],
        out_shape=[
            jax.ShapeDtypeStruct((2, NP, F), f32),
            jax.ShapeDtypeStruct((NP, 1), f32),
        ],
    )(degp3, xp, W1)

    a1 = _agg2_sc(h1p.reshape(2 * NP, F), row2, row2b, col2, zeros)

    h2p = pl.pallas_call(
        _tc_mid_body,
        grid=(GN,),
        in_specs=[
            pl.BlockSpec((2, NC, BN, F), lambda n: (0, 0, n, 0)),
            pl.BlockSpec((2, BN, F), lambda n: (0, n, 0)),
            pl.BlockSpec((BN, 1), lambda n: (n, 0)),
            pl.BlockSpec((2, 128), lambda n: (0, 0)),
            pl.BlockSpec((256, 128), lambda n: (0, 0)),
        ],
        out_specs=pl.BlockSpec((BN, F), lambda n: (n, 0)),
        out_shape=jax.ShapeDtypeStruct((NP, F), f32),
    )(a1, h1p, dinv, b1.reshape(2, 128), W2)

    a2 = _agg_sc(h2p, row2, col2, zeros)

    outp = pl.pallas_call(
        _tc_back_body,
        grid=(GN,),
        in_specs=[
            pl.BlockSpec((NC, BN, F), lambda n: (0, n, 0)),
            pl.BlockSpec((BN, F), lambda n: (n, 0)),
            pl.BlockSpec((BN, 1), lambda n: (n, 0)),
            pl.BlockSpec((1, 128), lambda n: (0, 0)),
        ],
        out_specs=pl.BlockSpec((BN, F), lambda n: (n, 0)),
        out_shape=jax.ShapeDtypeStruct((NP, F), f32),
    )(a2, h2p, dinv, b2.reshape(1, 128))

    return outp[:N]


# speedup vs baseline: 1.0124x; 1.0124x over previous
import functools

import jax
import jax.numpy as jnp
from jax import lax
from jax.experimental import pallas as pl
from jax.experimental.pallas import tpu as pltpu
from jax.experimental.pallas import tpu_sc as plsc

N = 10000
NP = 10240
E = 320000
EP2 = 327680
NC = 2
NS = 16
NW = NC * NS
EPT = EP2 // NW
CH = 64
NCHT = EPT // CH
NCHR = NCHT // 4
NBUF = 4
PT = NP // NS
BN = 2000
GN = N // BN
F = 128

def _deg_sc_body(col_hbm, out_hbm, colv, hist, shared, accin, outv):
    c = lax.axis_index("c")
    s = lax.axis_index("s")
    w = c * NS + s
    pltpu.sync_copy(col_hbm.at[pl.ds(w * NCHT, NCHT)], colv)

    def zero_body(i, _):
        hist[pl.ds(i * 16, 16)] = jnp.zeros((16,), jnp.float32)
        return 0
    lax.fori_loop(0, NP // 16, zero_body, 0)

    ones = jnp.ones((16,), jnp.float32)

    def hist_body(i, _):
        r = i // (CH // 16)
        k = (i % (CH // 16)) * 16
        idx = colv[r, pl.ds(k, 16)]
        plsc.addupdate_scatter(hist, [idx], ones)
        return 0
    lax.fori_loop(0, EPT // 16, hist_body, 0)

    pltpu.sync_copy(hist, shared.at[s])
    plsc.subcore_barrier()
    pltpu.sync_copy(shared.at[:, pl.ds(s * PT, PT)], accin)

    def red_body(k, _):
        acc = accin[0, pl.ds(k * 16, 16)]
        for t in range(1, NS):
            acc = acc + accin[t, pl.ds(k * 16, 16)]
        outv[pl.ds(k * 16, 16)] = acc
        return 0
    lax.fori_loop(0, PT // 16, red_body, 0)
    pltpu.sync_copy(outv, out_hbm.at[c, pl.ds(s * PT, PT)])


def _agg_pass(table_hbm, row_hbm, col_hbm, zeros_hbm, dump,
              rowi, coli, bufs, acc, gsems, ssems, s, w):
    pltpu.sync_copy(zeros_hbm.at[pl.ds(s * PT, PT)], acc.at[pl.ds(s * PT, PT)])
    plsc.subcore_barrier()

    def gstart(jj, b):
        pltpu.async_copy(table_hbm.at[rowi.at[jj]], bufs[b], gsems[b])

    def gwait(jj, b):
        pltpu.make_async_copy(table_hbm.at[rowi.at[jj]], bufs[b],
                              gsems[b]).wait()

    def sstart(jj, b):
        pltpu.async_copy(bufs[b], acc.at[coli.at[jj]], ssems[b], add=True)

    def swait(jj, b):
        pltpu.make_async_copy(bufs[b], acc.at[coli.at[jj]], ssems[b]).wait()

    for h in range(NCHT // NCHR):
        base = w * NCHT + h * NCHR
        pltpu.sync_copy(row_hbm.at[pl.ds(base, NCHR)], rowi)
        pltpu.sync_copy(col_hbm.at[pl.ds(base, NCHR)], coli)
        for b in range(NBUF):
            gstart(b, b)

        def body(i, _):
            for b in range(NBUF):
                jj = NBUF * i + b
                gwait(jj, b)
                sstart(jj, b)
            for b in range(NBUF):
                jj = NBUF * i + b

                @pl.when(jj + NBUF < NCHR)
                def _():
                    swait(jj, b)
                    gstart(jj + NBUF, b)
            return 0
        lax.fori_loop(0, NCHR // NBUF, body, 0)
        for b in range(NBUF):
            swait(NCHR - NBUF + b, b)

    plsc.subcore_barrier()
    dump()


def _unpack_scratch(rest):
    bufs = rest[:NBUF]
    acc = rest[NBUF]
    gsems = rest[NBUF + 1:2 * NBUF + 1]
    ssems = rest[2 * NBUF + 1:]
    return bufs, acc, gsems, ssems


def _agg_sc_body(table_hbm, row_hbm, col_hbm, zeros_hbm, out_hbm,
                 rowi, coli, *rest):
    c = lax.axis_index("c")
    s = lax.axis_index("s")
    w = c * NS + s
    bufs, acc, gsems, ssems = _unpack_scratch(rest)

    def dump():
        pltpu.sync_copy(acc.at[pl.ds(s * PT, PT)],
                        out_hbm.at[c, pl.ds(s * PT, PT)])
    _agg_pass(table_hbm, row_hbm, col_hbm, zeros_hbm, dump,
              rowi, coli, bufs, acc, gsems, ssems, s, w)


def _agg2_sc_body(table_hbm, row0_hbm, row1_hbm, col_hbm, zeros_hbm, out_hbm,
                  rowi, coli, *rest):
    c = lax.axis_index("c")
    s = lax.axis_index("s")
    w = c * NS + s
    bufs, acc, gsems, ssems = _unpack_scratch(rest)

    for f, row_hbm in enumerate((row0_hbm, row1_hbm)):
        def dump(f=f):
            pltpu.sync_copy(acc.at[pl.ds(s * PT, PT)],
                            out_hbm.at[f, c, pl.ds(s * PT, PT)])
        _agg_pass(table_hbm, row_hbm, col_hbm, zeros_hbm, dump,
                  rowi, coli, bufs, acc, gsems, ssems, s, w)


@functools.lru_cache(maxsize=None)
def _sc_kernels():
    mesh = plsc.VectorSubcoreMesh(
        core_axis_name="c", subcore_axis_name="s",
        num_cores=NC, num_subcores=NS)
    params = pltpu.CompilerParams(needs_layout_passes=False)
    deg = pl.kernel(
        _deg_sc_body,
        out_type=jax.ShapeDtypeStruct((NC, NP), jnp.float32),
        mesh=mesh,
        compiler_params=params,
        scratch_types=[
            pltpu.VMEM((NCHT, CH), jnp.int32),
            pltpu.VMEM((NP,), jnp.float32),
            pltpu.VMEM_SHARED((NS, NP), jnp.float32),
            pltpu.VMEM((NS, PT), jnp.float32),
            pltpu.VMEM((PT,), jnp.float32),
        ],
    )
    agg_scratch = (
        [
            pltpu.VMEM((NCHR, CH), jnp.int32),
            pltpu.VMEM((NCHR, CH), jnp.int32),
        ]
        + [pltpu.VMEM((CH, F), jnp.float32)] * NBUF
        + [pltpu.VMEM_SHARED((NP, F), jnp.float32)]
        + [pltpu.SemaphoreType.DMA] * (2 * NBUF)
    )
    agg = pl.kernel(
        _agg_sc_body,
        out_type=jax.ShapeDtypeStruct((NC, NP, F), jnp.float32),
        mesh=mesh,
        compiler_params=params,
        scratch_types=agg_scratch,
    )
    agg2 = pl.kernel(
        _agg2_sc_body,
        out_type=jax.ShapeDtypeStruct((2, NC, NP, F), jnp.float32),
        mesh=mesh,
        compiler_params=params,
        scratch_types=agg_scratch,
    )
    return deg, agg, agg2


_SLOPE = (1.0 / 8.0 + 1.0 / 3.0) / 2.0


def _tc_front_body(deg_ref, x_ref, w1_ref, hp_ref, dinv_ref):
    deg = deg_ref[0] + deg_ref[1] + 1.0
    dinv = lax.rsqrt(deg)
    h = jnp.dot(x_ref[...], w1_ref[...], preferred_element_type=jnp.float32)
    hp = h * dinv
    hp_ref[0] = hp[:, :F]
    hp_ref[1] = hp[:, F:]
    dinv_ref[...] = dinv


def _tc_mid_body(a_ref, hp_ref, dinv_ref, b1_ref, w2_ref, h2_ref):
    dinv = dinv_ref[...]
    u0 = dinv * (a_ref[0, 0] + a_ref[0, 1] + hp_ref[0]) + b1_ref[0:1, :]
    u1 = dinv * (a_ref[1, 0] + a_ref[1, 1] + hp_ref[1]) + b1_ref[1:2, :]
    act0 = jnp.where(u0 >= 0, u0, _SLOPE * u0)
    act1 = jnp.where(u1 >= 0, u1, _SLOPE * u1)
    h2 = (jnp.dot(act0, w2_ref[:F, :], preferred_element_type=jnp.float32)
          + jnp.dot(act1, w2_ref[F:, :], preferred_element_type=jnp.float32))
    h2_ref[...] = h2 * dinv


def _tc_back_body(a2_ref, h2_ref, dinv_ref, b2_ref, out_ref):
    out_ref[...] = (dinv_ref[...] * (a2_ref[0] + a2_ref[1] + h2_ref[...])
                    + b2_ref[0:1, :])


def kernel(x, edge_index, W1, b1, W2, b2):
    f32 = jnp.float32
    _deg_sc, _agg_sc, _agg2_sc = _sc_kernels()
    pidx = jnp.arange(EP2 - E, dtype=jnp.int32)
    rowcat = jnp.concatenate([edge_index[0], pidx % N])
    row2 = rowcat.reshape(EP2 // CH, CH)
    row2b = (rowcat + N).reshape(EP2 // CH, CH)
    col2 = jnp.concatenate([edge_index[1], N + pidx % (NP - N)]).reshape(
        EP2 // CH, CH)
    zeros = jnp.zeros((NP, F), f32)

    degp = _deg_sc(col2)
    degp3 = degp.reshape(NC, NP, 1)

    h1p, dinv = pl.pallas_call(
        _tc_front_body,
        grid=(GN,),
        in_specs=[
            pl.BlockSpec((NC, BN, 1), lambda n: (0, n, 0)),
            pl.BlockSpec((BN, 128), lambda n: (n, 0)),
            pl.BlockSpec((128, 256), lambda n: (0, 0)),
        ],
        out_specs=[
            pl.BlockSpec((2, BN, F), lambda n: (0, n, 0)),
            pl.BlockSpec((BN, 1), lambda n: (n, 0)),
        ],
        out_shape=[
            jax.ShapeDtypeStruct((2, N, F), f32),
            jax.ShapeDtypeStruct((N, 1), f32),
        ],
    )(degp3, x, W1)

    a1 = _agg2_sc(h1p.reshape(2 * N, F), row2, row2b, col2, zeros)

    h2p = pl.pallas_call(
        _tc_mid_body,
        grid=(GN,),
        in_specs=[
            pl.BlockSpec((2, NC, BN, F), lambda n: (0, 0, n, 0)),
            pl.BlockSpec((2, BN, F), lambda n: (0, n, 0)),
            pl.BlockSpec((BN, 1), lambda n: (n, 0)),
            pl.BlockSpec((2, 128), lambda n: (0, 0)),
            pl.BlockSpec((256, 128), lambda n: (0, 0)),
        ],
        out_specs=pl.BlockSpec((BN, F), lambda n: (n, 0)),
        out_shape=jax.ShapeDtypeStruct((N, F), f32),
    )(a1, h1p, dinv, b1.reshape(2, 128), W2)

    a2 = _agg_sc(h2p, row2, col2, zeros)

    outp = pl.pallas_call(
        _tc_back_body,
        grid=(GN,),
        in_specs=[
            pl.BlockSpec((NC, BN, F), lambda n: (0, n, 0)),
            pl.BlockSpec((BN, F), lambda n: (n, 0)),
            pl.BlockSpec((BN, 1), lambda n: (n, 0)),
            pl.BlockSpec((1, 128), lambda n: (0, 0)),
        ],
        out_specs=pl.BlockSpec((BN, F), lambda n: (n, 0)),
        out_shape=jax.ShapeDtypeStruct((N, F), f32),
    )(a2, h2p, dinv, b2.reshape(1, 128))

    return outp
